# baseline (speedup 1.0000x reference)
import functools

import jax
import jax.numpy as jnp
from jax import lax
from jax.experimental import pallas as pl
from jax.experimental.pallas import tpu as pltpu
from jax.experimental.pallas import tpu_sc as plsc

N = 10000
E = 320000
F = 128
M = 256
EA = 16
CD = 4
CDP = 8
HALF = M // 2

NC, NS = 2, 16
NW = NC * NS
NP = 10112
XACC = 40960
XB = 8192
CH = 128
NCHUNK = E // CH

EB = 1280
G = E // EB
NB = 2000
GN = N // NB

_f32 = jnp.float32


def _silu(v):
    return v * jax.nn.sigmoid(v)


def _mesh():
    return plsc.VectorSubcoreMesh(
        core_axis_name="c", subcore_axis_name="s", num_cores=NC, num_subcores=NS
    )


def _pre_body(h_ref, ab_ref, ha_ref, hb_ref):
    res = jnp.dot(h_ref[...], ab_ref[...], preferred_element_type=_f32)
    ha_ref[...] = res[:, :M]
    hb_ref[...] = res[:, M:]


def _p_pre(h, ab):
    return pl.pallas_call(
        _pre_body,
        grid=(GN,),
        in_specs=[
            pl.BlockSpec((NB, F), lambda i: (i, 0)),
            pl.BlockSpec((F, 2 * M), lambda i: (0, 0)),
        ],
        out_specs=[
            pl.BlockSpec((NB, M), lambda i: (i, 0)),
            pl.BlockSpec((NB, M), lambda i: (i, 0)),
        ],
        out_shape=[
            jax.ShapeDtypeStruct((N, M), _f32),
            jax.ShapeDtypeStruct((N, M), _f32),
        ],
    )(h, ab)


def _gather_body(ha, hb, x4, row, col, ga, gb, cd8,
                 idx_r, idx_c, x_vmem, ga_buf, gb_buf, cd_buf,
                 sem_a, sem_b):
    cid = lax.axis_index("c")
    sid = lax.axis_index("s")
    wid = sid * NC + cid

    pltpu.sync_copy(x4, x_vmem)

    def body(i, carry):
        c = wid + i * NW

        @pl.when(c < NCHUNK)
        def _():
            e0 = c * CH
            pltpu.sync_copy(row.at[pl.ds(e0, CH)], idx_r)
            pltpu.sync_copy(col.at[pl.ds(e0, CH)], idx_c)
            cpa = pltpu.async_copy(ha.at[idx_r], ga_buf, sem_a)
            cpb = pltpu.async_copy(hb.at[idx_c], gb_buf, sem_b)

            def gbody(g, carry2):
                base = g * 16
                l8 = (lax.iota(jnp.int32, 16) + base) * CDP
                rv = idx_r[pl.ds(base, 16)] * CD
                cv = idx_c[pl.ds(base, 16)] * CD
                acc = jnp.zeros((16,), _f32)
                for cc in range(CD):
                    xr = plsc.load_gather(x_vmem, [rv + cc])
                    xc = plsc.load_gather(x_vmem, [cv + cc])
                    d = xr - xc
                    plsc.store_scatter(cd_buf, [l8 + cc], d)
                    acc = acc + d * d
                plsc.store_scatter(cd_buf, [l8 + CD], acc)
                return carry2

            lax.fori_loop(0, CH // 16, gbody, 0)
            cpa.wait()
            cpb.wait()
            pltpu.sync_copy(ga_buf, ga.at[pl.ds(e0, CH)])
            pltpu.sync_copy(gb_buf, gb.at[pl.ds(e0, CH)])
            pltpu.sync_copy(cd_buf, cd8.at[pl.ds(e0 * CDP, CH * CDP)])

        return carry

    lax.fori_loop(0, (NCHUNK + NW - 1) // NW, body, 0)


def _p_gather(ha, hb, x4, row, col):
    k = functools.partial(
        pl.kernel,
        out_type=[
            jax.ShapeDtypeStruct((E, HALF), _f32),
            jax.ShapeDtypeStruct((E, HALF), _f32),
            jax.ShapeDtypeStruct((E * CDP,), _f32),
        ],
        mesh=_mesh(),
        scratch_types=[
            pltpu.VMEM((CH,), jnp.int32),
            pltpu.VMEM((CH,), jnp.int32),
            pltpu.VMEM((N * CD,), _f32),
            pltpu.VMEM((CH, HALF), _f32),
            pltpu.VMEM((CH, HALF), _f32),
            pltpu.VMEM((CH * CDP,), _f32),
            pltpu.SemaphoreType.DMA,
            pltpu.SemaphoreType.DMA,
        ],
        compiler_params=pltpu.CompilerParams(needs_layout_passes=False),
    )
    return k(_gather_body)(ha, hb, x4, row, col)


def _edge_body(ga, gb, cd8, ea, dmat, wrv, be1v, we2, be2v, wc1, bc1v,
               wc2t, msg2, cw):
    rad = cd8[:, CD:CD + 1]
    ua = lax.bitcast_convert_type(ga[...], jnp.int32)
    ub = lax.bitcast_convert_type(gb[...], jnp.int32)
    hi_mask = jnp.int32(-65536)
    s_e = (lax.bitcast_convert_type(jnp.left_shift(ua, 16), _f32)
           + lax.bitcast_convert_type(jnp.left_shift(ub, 16), _f32))
    s_o = (lax.bitcast_convert_type(ua & hi_mask, _f32)
           + lax.bitcast_convert_type(ub & hi_mask, _f32))
    pre = (jnp.concatenate([s_e, s_o], axis=1)
           + jnp.dot(ea[...], dmat[...], preferred_element_type=_f32)
           + rad * wrv[...] + be1v[...])
    m1 = _silu(pre).astype(jnp.bfloat16)
    msg = _silu(jnp.dot(m1, we2[...], preferred_element_type=_f32) + be2v[...])
    msg2[0, :, :] = msg[:, :HALF]
    msg2[1, :, :] = msg[:, HALF:]
    msgb = msg.astype(jnp.bfloat16)
    t = _silu(jnp.dot(msgb, wc1[...], preferred_element_type=_f32) + bc1v[...])
    cw[0] = lax.dot_general(wc2t[...], t, (((1,), (1,)), ((), ())),
                            preferred_element_type=_f32)


def _p_edge(ga, gb, cd8, ea, dmat, wrv, be1v, we2, be2v, wc1, bc1v, wc2t):
    full = lambda shape: pl.BlockSpec(shape, lambda i: tuple(0 for _ in shape))
    return pl.pallas_call(
        _edge_body,
        grid=(G,),
        in_specs=[
            pl.BlockSpec((EB, HALF), lambda i: (i, 0)),
            pl.BlockSpec((EB, HALF), lambda i: (i, 0)),
            pl.BlockSpec((EB, CDP), lambda i: (i, 0)),
            pl.BlockSpec((EB, EA), lambda i: (i, 0)),
            full((EA, M)),
            full((1, M)),
            full((1, M)),
            full((M, M)),
            full((1, M)),
            full((M, M)),
            full((1, M)),
            full((1, M)),
        ],
        out_specs=[
            pl.BlockSpec((2, EB, HALF), lambda i: (0, i, 0)),
            pl.BlockSpec((1, 1, EB), lambda i: (i, 0, 0)),
        ],
        out_shape=[
            jax.ShapeDtypeStruct((2, E, HALF), _f32),
            jax.ShapeDtypeStruct((G, 1, EB), _f32),
        ],
    )(ga, gb, cd8, ea, dmat, wrv, be1v, we2, be2v, wc1, bc1v, wc2t)


def _scatter_msg_body(msgf, row, z128, magg2, acc, idx_m, msg_buf):
    cid = lax.axis_index("c")
    sid = lax.axis_index("s")
    rpt = NP // NS

    pltpu.sync_copy(z128.at[pl.ds(sid * rpt, rpt)],
                    acc.at[pl.ds(sid * rpt, rpt)])

    plsc.subcore_barrier()

    def mbody(i, carry):
        c = sid + i * NS

        @pl.when(c < NCHUNK)
        def _():
            e0 = c * CH
            pltpu.sync_copy(row.at[pl.ds(e0, CH)], idx_m)
            pltpu.sync_copy(msgf.at[pl.ds(cid * E + e0, CH)], msg_buf)
            pltpu.sync_copy(msg_buf, acc.at[idx_m], add=True)

        return carry

    lax.fori_loop(0, (NCHUNK + NS - 1) // NS, mbody, 0)

    plsc.subcore_barrier()

    pltpu.sync_copy(acc.at[pl.ds(sid * rpt, rpt)],
                    magg2.at[pl.ds(cid * NP + sid * rpt, rpt)])


def _p_scatter_msg(msgf, row, z128):
    k = functools.partial(
        pl.kernel,
        out_type=jax.ShapeDtypeStruct((2 * NP, HALF), _f32),
        mesh=_mesh(),
        scratch_types=[
            pltpu.VMEM_SHARED((NP, HALF), _f32),
            pltpu.VMEM((CH,), jnp.int32),
            pltpu.VMEM((CH, HALF), _f32),
        ],
        compiler_params=pltpu.CompilerParams(needs_layout_passes=False),
    )
    return k(_scatter_msg_body)(msgf, row, z128)


def _scatter_x_body(cwf, cd8, row, zx, xp32, idx_w, cw_buf, cd_buf, accx):
    cid = lax.axis_index("c")
    sid = lax.axis_index("s")
    wid = sid * NC + cid

    pltpu.sync_copy(zx, accx)

    def wbody(i, carry):
        c = wid + i * NW

        @pl.when(c < NCHUNK)
        def _():
            e0 = c * CH
            pltpu.sync_copy(row.at[pl.ds(e0, CH)], idx_w)
            pltpu.sync_copy(cwf.at[pl.ds(e0, CH)], cw_buf)
            pltpu.sync_copy(cd8.at[pl.ds(e0 * CDP, CH * CDP)], cd_buf)

            def gbody(g, carry2):
                base = g * 16
                l8 = (lax.iota(jnp.int32, 16) + base) * CDP
                rv = idx_w[pl.ds(base, 16)] * CD
                cwv = cw_buf[pl.ds(base, 16)]
                for cc in range(CD):
                    dv = plsc.load_gather(cd_buf, [l8 + cc])
                    plsc.addupdate_scatter(accx, [rv + cc], dv * cwv)
                return carry2

            lax.fori_loop(0, CH // 16, gbody, 0)

        return carry

    lax.fori_loop(0, (NCHUNK + NW - 1) // NW, wbody, 0)

    for g in range(XACC // XB):
        pltpu.sync_copy(accx.at[pl.ds(g * XB, XB)],
                        xp32.at[pl.ds((g * NW + wid) * XB, XB)])


def _p_scatter_x(cwf, cd8, row, zx):
    k = functools.partial(
        pl.kernel,
        out_type=jax.ShapeDtypeStruct((NW * XACC,), _f32),
        mesh=_mesh(),
        scratch_types=[
            pltpu.VMEM((CH,), jnp.int32),
            pltpu.VMEM((CH,), _f32),
            pltpu.VMEM((CH * CDP,), _f32),
            pltpu.VMEM((XACC,), _f32),
        ],
        compiler_params=pltpu.CompilerParams(needs_layout_passes=False),
    )
    return k(_scatter_x_body)(cwf, cd8, row, zx)


def _final_body(h_ref, ml, mr, xv, xps, wh, wml, wmr, bf1v, wf2, bf2v,
                out, xout):
    hf = (jnp.dot(h_ref[...], wh[...], preferred_element_type=_f32)
          + jnp.dot(ml[...], wml[...], preferred_element_type=_f32)
          + jnp.dot(mr[...], wmr[...], preferred_element_type=_f32)
          + bf1v[...])
    hf = _silu(hf)
    out[...] = jnp.dot(hf, wf2[...], preferred_element_type=_f32) + bf2v[...]
    xout[0, 0, :] = xv[0, 0, :] + jnp.sum(xps[0], axis=0)


def _p_final(h, ml, mr, xflat, xp32, wh, wml, wmr, bf1v, wf2, bf2v):
    full = lambda shape: pl.BlockSpec(shape, lambda i: tuple(0 for _ in shape))
    return pl.pallas_call(
        _final_body,
        grid=(GN,),
        in_specs=[
            pl.BlockSpec((NB, F), lambda i: (i, 0)),
            pl.BlockSpec((NB, HALF), lambda i: (i, 0)),
            pl.BlockSpec((NB, HALF), lambda i: (i, 0)),
            pl.BlockSpec((1, 1, XB), lambda i: (i, 0, 0)),
            pl.BlockSpec((1, NW, XB), lambda i: (i, 0, 0)),
            full((F, M)),
            full((HALF, M)),
            full((HALF, M)),
            full((1, M)),
            full((M, F)),
            full((1, F)),
        ],
        out_specs=[
            pl.BlockSpec((NB, F), lambda i: (i, 0)),
            pl.BlockSpec((1, 1, XB), lambda i: (i, 0, 0)),
        ],
        out_shape=[
            jax.ShapeDtypeStruct((N, F), _f32),
            jax.ShapeDtypeStruct((GN, 1, XB), _f32),
        ],
    )(h, ml, mr, xflat, xp32, wh, wml, wmr, bf1v, wf2, bf2v)


def kernel(h, x, edge_index, edge_attribute, We1, be1, We2, be2,
           Wf1, bf1, Wf2, bf2, Wc1, bc1, Wc2):
    row = edge_index[0].astype(jnp.int32)
    col = edge_index[1].astype(jnp.int32)

    ab = jnp.concatenate([We1[:F], We1[F:2 * F]], axis=1)
    perm = jnp.concatenate([jnp.arange(0, M, 2), jnp.arange(1, M, 2)])
    wrv = We1[2 * F][perm].reshape(1, M)
    dmat = We1[2 * F + 1:][:, perm]

    ha, hb = _p_pre(h, ab)
    pack = lambda t: lax.bitcast_convert_type(
        t.astype(jnp.bfloat16).reshape(N, HALF, 2), _f32)
    ga, gb, cd8 = _p_gather(pack(ha), pack(hb), x.reshape(N * CD), row, col)

    msg2, cwg = _p_edge(
        ga, gb, cd8.reshape(E, CDP), edge_attribute, dmat, wrv,
        be1[perm].reshape(1, M), We2[perm].astype(jnp.bfloat16),
        be2.reshape(1, M), Wc1.astype(jnp.bfloat16),
        bc1.reshape(1, M), Wc2.reshape(1, M))

    msgf = msg2.reshape(2 * E, HALF)
    cwf = cwg.reshape(E)
    magg2 = _p_scatter_msg(msgf, row, jnp.zeros((NP, HALF), _f32))
    xp32 = _p_scatter_x(cwf, cd8, row, jnp.zeros((XACC,), _f32))

    xflat = jnp.pad(x.reshape(N * CD), (0, XACC - N * CD)).reshape(GN, 1, XB)
    h_updated, xf = _p_final(
        h, magg2[:N], magg2[NP:NP + N], xflat, xp32.reshape(GN, NW, XB),
        Wf1[:F], Wf1[F:F + HALF], Wf1[F + HALF:], bf1.reshape(1, M), Wf2,
        bf2.reshape(1, F))
    x_updated = xf.reshape(XACC)[:N * CD].reshape(N, CD)
    return (h_updated, x_updated)


# speedup vs baseline: 1.1379x; 1.1379x over previous
import functools

import jax
import jax.numpy as jnp
from jax import lax
from jax.experimental import pallas as pl
from jax.experimental.pallas import tpu as pltpu
from jax.experimental.pallas import tpu_sc as plsc

N = 10000
E = 320000
F = 128
M = 256
EA = 16
CD = 4
CDP = 8
HALF = M // 2

NC, NS = 2, 16
NW = NC * NS
NP = 10112
XACC = 40960
XB = 8192
CH = 128

Q = 5
EQ = E // Q
NCHUNK_Q = EQ // CH

EB = 1280
G_Q = EQ // EB
NB = 2000
GN = N // NB

_f32 = jnp.float32


def _silu(v):
    return v * jax.nn.sigmoid(v)


def _mesh():
    return plsc.VectorSubcoreMesh(
        core_axis_name="c", subcore_axis_name="s", num_cores=NC, num_subcores=NS
    )


def _pre_body(h_ref, ab_ref, ha_ref, hb_ref):
    res = jnp.dot(h_ref[...], ab_ref[...], preferred_element_type=_f32)
    ha_ref[...] = res[:, :M]
    hb_ref[...] = res[:, M:]


def _p_pre(h, ab):
    return pl.pallas_call(
        _pre_body,
        grid=(GN,),
        in_specs=[
            pl.BlockSpec((NB, F), lambda i: (i, 0)),
            pl.BlockSpec((F, 2 * M), lambda i: (0, 0)),
        ],
        out_specs=[
            pl.BlockSpec((NB, M), lambda i: (i, 0)),
            pl.BlockSpec((NB, M), lambda i: (i, 0)),
        ],
        out_shape=[
            jax.ShapeDtypeStruct((N, M), _f32),
            jax.ShapeDtypeStruct((N, M), _f32),
        ],
    )(h, ab)


def _gather_body(ha, hb, x4, row, col, ga, gb, cd8,
                 idx_r, idx_c, x_vmem, ga_buf, gb_buf, cd_buf,
                 sem_a, sem_b):
    cid = lax.axis_index("c")
    sid = lax.axis_index("s")
    wid = sid * NC + cid

    pltpu.sync_copy(x4, x_vmem)

    def body(i, carry):
        c = wid + i * NW

        @pl.when(c < NCHUNK_Q)
        def _():
            e0 = c * CH
            pltpu.sync_copy(row.at[pl.ds(e0, CH)], idx_r)
            pltpu.sync_copy(col.at[pl.ds(e0, CH)], idx_c)
            cpa = pltpu.async_copy(ha.at[idx_r], ga_buf, sem_a)
            cpb = pltpu.async_copy(hb.at[idx_c], gb_buf, sem_b)

            def gbody(g, carry2):
                base = g * 16
                l8 = (lax.iota(jnp.int32, 16) + base) * CDP
                rv = idx_r[pl.ds(base, 16)] * CD
                cv = idx_c[pl.ds(base, 16)] * CD
                acc = jnp.zeros((16,), _f32)
                for cc in range(CD):
                    xr = plsc.load_gather(x_vmem, [rv + cc])
                    xc = plsc.load_gather(x_vmem, [cv + cc])
                    d = xr - xc
                    plsc.store_scatter(cd_buf, [l8 + cc], d)
                    acc = acc + d * d
                plsc.store_scatter(cd_buf, [l8 + CD], acc)
                return carry2

            lax.fori_loop(0, CH // 16, gbody, 0)
            cpa.wait()
            cpb.wait()
            pltpu.sync_copy(ga_buf, ga.at[pl.ds(e0, CH)])
            pltpu.sync_copy(gb_buf, gb.at[pl.ds(e0, CH)])
            pltpu.sync_copy(cd_buf, cd8.at[pl.ds(e0 * CDP, CH * CDP)])

        return carry

    lax.fori_loop(0, (NCHUNK_Q + NW - 1) // NW, body, 0)


def _p_gather(ha, hb, x4, row, col):
    k = functools.partial(
        pl.kernel,
        out_type=[
            jax.ShapeDtypeStruct((EQ, HALF), _f32),
            jax.ShapeDtypeStruct((EQ, HALF), _f32),
            jax.ShapeDtypeStruct((EQ * CDP,), _f32),
        ],
        mesh=_mesh(),
        scratch_types=[
            pltpu.VMEM((CH,), jnp.int32),
            pltpu.VMEM((CH,), jnp.int32),
            pltpu.VMEM((N * CD,), _f32),
            pltpu.VMEM((CH, HALF), _f32),
            pltpu.VMEM((CH, HALF), _f32),
            pltpu.VMEM((CH * CDP,), _f32),
            pltpu.SemaphoreType.DMA,
            pltpu.SemaphoreType.DMA,
        ],
        compiler_params=pltpu.CompilerParams(needs_layout_passes=False),
    )
    return k(_gather_body)(ha, hb, x4, row, col)


def _edge_body(ga, gb, cd8, ea, dmat, wrv, be1v, we2, be2v, wc1, bc1v,
               wc2t, msg2, cw):
    rad = cd8[:, CD:CD + 1]
    ua = lax.bitcast_convert_type(ga[...], jnp.int32)
    ub = lax.bitcast_convert_type(gb[...], jnp.int32)
    hi_mask = jnp.int32(-65536)
    s_e = (lax.bitcast_convert_type(jnp.left_shift(ua, 16), _f32)
           + lax.bitcast_convert_type(jnp.left_shift(ub, 16), _f32))
    s_o = (lax.bitcast_convert_type(ua & hi_mask, _f32)
           + lax.bitcast_convert_type(ub & hi_mask, _f32))
    pre = (jnp.concatenate([s_e, s_o], axis=1)
           + jnp.dot(ea[...], dmat[...], preferred_element_type=_f32)
           + rad * wrv[...] + be1v[...])
    m1 = _silu(pre).astype(jnp.bfloat16)
    msg = _silu(jnp.dot(m1, we2[...], preferred_element_type=_f32) + be2v[...])
    msg2[0, :, :] = msg[:, :HALF]
    msg2[1, :, :] = msg[:, HALF:]
    msgb = msg.astype(jnp.bfloat16)
    t = _silu(jnp.dot(msgb, wc1[...], preferred_element_type=_f32) + bc1v[...])
    cw[0] = lax.dot_general(wc2t[...], t, (((1,), (1,)), ((), ())),
                            preferred_element_type=_f32)


def _p_edge(ga, gb, cd8, ea, dmat, wrv, be1v, we2, be2v, wc1, bc1v, wc2t):
    full = lambda shape: pl.BlockSpec(shape, lambda i: tuple(0 for _ in shape))
    return pl.pallas_call(
        _edge_body,
        grid=(G_Q,),
        in_specs=[
            pl.BlockSpec((EB, HALF), lambda i: (i, 0)),
            pl.BlockSpec((EB, HALF), lambda i: (i, 0)),
            pl.BlockSpec((EB, CDP), lambda i: (i, 0)),
            pl.BlockSpec((EB, EA), lambda i: (i, 0)),
            full((EA, M)),
            full((1, M)),
            full((1, M)),
            full((M, M)),
            full((1, M)),
            full((M, M)),
            full((1, M)),
            full((1, M)),
        ],
        out_specs=[
            pl.BlockSpec((2, EB, HALF), lambda i: (0, i, 0)),
            pl.BlockSpec((1, 1, EB), lambda i: (i, 0, 0)),
        ],
        out_shape=[
            jax.ShapeDtypeStruct((2, EQ, HALF), _f32),
            jax.ShapeDtypeStruct((G_Q, 1, EB), _f32),
        ],
    )(ga, gb, cd8, ea, dmat, wrv, be1v, we2, be2v, wc1, bc1v, wc2t)


def _scatter_msg_body(msgf, row, prev, magg2, acc, idx_m, msg_buf):
    cid = lax.axis_index("c")
    sid = lax.axis_index("s")
    rpt = NP // NS

    pltpu.sync_copy(prev.at[pl.ds(cid * NP + sid * rpt, rpt)],
                    acc.at[pl.ds(sid * rpt, rpt)])

    plsc.subcore_barrier()

    def mbody(i, carry):
        c = sid + i * NS

        @pl.when(c < NCHUNK_Q)
        def _():
            e0 = c * CH
            pltpu.sync_copy(row.at[pl.ds(e0, CH)], idx_m)
            pltpu.sync_copy(msgf.at[pl.ds(cid * EQ + e0, CH)], msg_buf)
            pltpu.sync_copy(msg_buf, acc.at[idx_m], add=True)

        return carry

    lax.fori_loop(0, (NCHUNK_Q + NS - 1) // NS, mbody, 0)

    plsc.subcore_barrier()

    pltpu.sync_copy(acc.at[pl.ds(sid * rpt, rpt)],
                    magg2.at[pl.ds(cid * NP + sid * rpt, rpt)])


def _p_scatter_msg(msgf, row, prev):
    k = functools.partial(
        pl.kernel,
        out_type=jax.ShapeDtypeStruct((2 * NP, HALF), _f32),
        mesh=_mesh(),
        scratch_types=[
            pltpu.VMEM_SHARED((NP, HALF), _f32),
            pltpu.VMEM((CH,), jnp.int32),
            pltpu.VMEM((CH, HALF), _f32),
        ],
        compiler_params=pltpu.CompilerParams(needs_layout_passes=False),
    )
    return k(_scatter_msg_body)(msgf, row, prev)


def _scatter_x_body(cwf, cd8, row, prev, xp32, idx_w, cw_buf, cd_buf, accx):
    cid = lax.axis_index("c")
    sid = lax.axis_index("s")
    wid = sid * NC + cid

    for g in range(XACC // XB):
        pltpu.sync_copy(prev.at[pl.ds((g * NW + wid) * XB, XB)],
                        accx.at[pl.ds(g * XB, XB)])

    def wbody(i, carry):
        c = wid + i * NW

        @pl.when(c < NCHUNK_Q)
        def _():
            e0 = c * CH
            pltpu.sync_copy(row.at[pl.ds(e0, CH)], idx_w)
            pltpu.sync_copy(cwf.at[pl.ds(e0, CH)], cw_buf)
            pltpu.sync_copy(cd8.at[pl.ds(e0 * CDP, CH * CDP)], cd_buf)

            def gbody(g, carry2):
                base = g * 16
                l8 = (lax.iota(jnp.int32, 16) + base) * CDP
                rv = idx_w[pl.ds(base, 16)] * CD
                cwv = cw_buf[pl.ds(base, 16)]
                for cc in range(CD):
                    dv = plsc.load_gather(cd_buf, [l8 + cc])
                    plsc.addupdate_scatter(accx, [rv + cc], dv * cwv)
                return carry2

            lax.fori_loop(0, CH // 16, gbody, 0)

        return carry

    lax.fori_loop(0, (NCHUNK_Q + NW - 1) // NW, wbody, 0)

    for g in range(XACC // XB):
        pltpu.sync_copy(accx.at[pl.ds(g * XB, XB)],
                        xp32.at[pl.ds((g * NW + wid) * XB, XB)])


def _p_scatter_x(cwf, cd8, row, prev):
    k = functools.partial(
        pl.kernel,
        out_type=jax.ShapeDtypeStruct((NW * XACC,), _f32),
        mesh=_mesh(),
        scratch_types=[
            pltpu.VMEM((CH,), jnp.int32),
            pltpu.VMEM((CH,), _f32),
            pltpu.VMEM((CH * CDP,), _f32),
            pltpu.VMEM((XACC,), _f32),
        ],
        compiler_params=pltpu.CompilerParams(needs_layout_passes=False),
    )
    return k(_scatter_x_body)(cwf, cd8, row, prev)


def _final_body(h_ref, ml, mr, xv, xps, wh, wml, wmr, bf1v, wf2, bf2v,
                out, xout):
    hf = (jnp.dot(h_ref[...], wh[...], preferred_element_type=_f32)
          + jnp.dot(ml[...], wml[...], preferred_element_type=_f32)
          + jnp.dot(mr[...], wmr[...], preferred_element_type=_f32)
          + bf1v[...])
    hf = _silu(hf)
    out[...] = jnp.dot(hf, wf2[...], preferred_element_type=_f32) + bf2v[...]
    xout[0, 0, :] = xv[0, 0, :] + jnp.sum(xps[0], axis=0)


def _p_final(h, ml, mr, xflat, xp32, wh, wml, wmr, bf1v, wf2, bf2v):
    full = lambda shape: pl.BlockSpec(shape, lambda i: tuple(0 for _ in shape))
    return pl.pallas_call(
        _final_body,
        grid=(GN,),
        in_specs=[
            pl.BlockSpec((NB, F), lambda i: (i, 0)),
            pl.BlockSpec((NB, HALF), lambda i: (i, 0)),
            pl.BlockSpec((NB, HALF), lambda i: (i, 0)),
            pl.BlockSpec((1, 1, XB), lambda i: (i, 0, 0)),
            pl.BlockSpec((1, NW, XB), lambda i: (i, 0, 0)),
            full((F, M)),
            full((HALF, M)),
            full((HALF, M)),
            full((1, M)),
            full((M, F)),
            full((1, F)),
        ],
        out_specs=[
            pl.BlockSpec((NB, F), lambda i: (i, 0)),
            pl.BlockSpec((1, 1, XB), lambda i: (i, 0, 0)),
        ],
        out_shape=[
            jax.ShapeDtypeStruct((N, F), _f32),
            jax.ShapeDtypeStruct((GN, 1, XB), _f32),
        ],
    )(h, ml, mr, xflat, xp32, wh, wml, wmr, bf1v, wf2, bf2v)


def kernel(h, x, edge_index, edge_attribute, We1, be1, We2, be2,
           Wf1, bf1, Wf2, bf2, Wc1, bc1, Wc2):
    row = edge_index[0].astype(jnp.int32)
    col = edge_index[1].astype(jnp.int32)

    ab = jnp.concatenate([We1[:F], We1[F:2 * F]], axis=1)
    perm = jnp.concatenate([jnp.arange(0, M, 2), jnp.arange(1, M, 2)])
    wrv = We1[2 * F][perm].reshape(1, M)
    dmat = We1[2 * F + 1:][:, perm]

    ha, hb = _p_pre(h, ab)
    pack = lambda t: lax.bitcast_convert_type(
        t.astype(jnp.bfloat16).reshape(N, HALF, 2), _f32)
    hap, hbp = pack(ha), pack(hb)
    xflat4 = x.reshape(N * CD)

    we2b = We2[perm].astype(jnp.bfloat16)
    wc1b = Wc1.astype(jnp.bfloat16)
    be1v = be1[perm].reshape(1, M)

    magg2 = jnp.zeros((2 * NP, HALF), _f32)
    xp32 = jnp.zeros((NW * XACC,), _f32)
    for q in range(Q):
        row_q = lax.slice(row, (q * EQ,), ((q + 1) * EQ,))
        col_q = lax.slice(col, (q * EQ,), ((q + 1) * EQ,))
        ea_q = lax.slice(edge_attribute, (q * EQ, 0), ((q + 1) * EQ, EA))
        ga, gb, cd8 = _p_gather(hap, hbp, xflat4, row_q, col_q)
        msg2, cwg = _p_edge(
            ga, gb, cd8.reshape(EQ, CDP), ea_q, dmat, wrv,
            be1v, we2b, be2.reshape(1, M), wc1b,
            bc1.reshape(1, M), Wc2.reshape(1, M))
        magg2 = _p_scatter_msg(msg2.reshape(2 * EQ, HALF), row_q, magg2)
        xp32 = _p_scatter_x(cwg.reshape(EQ), cd8, row_q, xp32)

    xflat = jnp.pad(x.reshape(N * CD), (0, XACC - N * CD)).reshape(GN, 1, XB)
    h_updated, xf = _p_final(
        h, magg2[:N], magg2[NP:NP + N], xflat, xp32.reshape(GN, NW, XB),
        Wf1[:F], Wf1[F:F + HALF], Wf1[F + HALF:], bf1.reshape(1, M), Wf2,
        bf2.reshape(1, F))
    x_updated = xf.reshape(XACC)[:N * CD].reshape(N, CD)
    return (h_updated, x_updated)


# speedup vs baseline: 1.1825x; 1.0391x over previous
import functools

import jax
import jax.numpy as jnp
from jax import lax
from jax.experimental import pallas as pl
from jax.experimental.pallas import tpu as pltpu
from jax.experimental.pallas import tpu_sc as plsc

N = 10000
E = 320000
F = 128
M = 256
EA = 16
CD = 4
CDP = 8
HALF = M // 2

NC, NS = 2, 16
NW = NC * NS
NP = 10112
XACC = 40960
XB = 8192
CH = 128
NCHUNK = E // CH

Q = 5
EQ = E // Q
NCHUNK_Q = EQ // CH

EB = 1280
G_Q = EQ // EB
NB = 2000
GN = N // NB

_f32 = jnp.float32


def _silu(v):
    return v * jax.nn.sigmoid(v)


def _mesh():
    return plsc.VectorSubcoreMesh(
        core_axis_name="c", subcore_axis_name="s", num_cores=NC, num_subcores=NS
    )


def _pre_body(h_ref, ab_ref, ha_ref, hb_ref):
    res = jnp.dot(h_ref[...], ab_ref[...], preferred_element_type=_f32)
    ha_ref[...] = res[:, :M]
    hb_ref[...] = res[:, M:]


def _p_pre(h, ab):
    return pl.pallas_call(
        _pre_body,
        grid=(GN,),
        in_specs=[
            pl.BlockSpec((NB, F), lambda i: (i, 0)),
            pl.BlockSpec((F, 2 * M), lambda i: (0, 0)),
        ],
        out_specs=[
            pl.BlockSpec((NB, M), lambda i: (i, 0)),
            pl.BlockSpec((NB, M), lambda i: (i, 0)),
        ],
        out_shape=[
            jax.ShapeDtypeStruct((N, M), _f32),
            jax.ShapeDtypeStruct((N, M), _f32),
        ],
    )(h, ab)


def _gather_body(ha, hb, x4, row, col, ga, gb, cd8,
                 idx_r, idx_c, x_vmem, ga_buf, gb_buf, cd_buf,
                 sem_a, sem_b):
    cid = lax.axis_index("c")
    sid = lax.axis_index("s")
    wid = sid * NC + cid

    pltpu.sync_copy(x4, x_vmem)

    def body(i, carry):
        c = wid + i * NW

        @pl.when(c < NCHUNK_Q)
        def _():
            e0 = c * CH
            pltpu.sync_copy(row.at[pl.ds(e0, CH)], idx_r)
            pltpu.sync_copy(col.at[pl.ds(e0, CH)], idx_c)
            cpa = pltpu.async_copy(ha.at[idx_r], ga_buf, sem_a)
            cpb = pltpu.async_copy(hb.at[idx_c], gb_buf, sem_b)

            def gbody(g, carry2):
                base = g * 16
                l8 = (lax.iota(jnp.int32, 16) + base) * CDP
                rv = idx_r[pl.ds(base, 16)] * CD
                cv = idx_c[pl.ds(base, 16)] * CD
                acc = jnp.zeros((16,), _f32)
                for cc in range(CD):
                    xr = plsc.load_gather(x_vmem, [rv + cc])
                    xc = plsc.load_gather(x_vmem, [cv + cc])
                    d = xr - xc
                    plsc.store_scatter(cd_buf, [l8 + cc], d)
                    acc = acc + d * d
                plsc.store_scatter(cd_buf, [l8 + CD], acc)
                return carry2

            lax.fori_loop(0, CH // 16, gbody, 0)
            cpa.wait()
            cpb.wait()
            pltpu.sync_copy(ga_buf, ga.at[pl.ds(e0, CH)])
            pltpu.sync_copy(gb_buf, gb.at[pl.ds(e0, CH)])
            pltpu.sync_copy(cd_buf, cd8.at[pl.ds(e0 * CDP, CH * CDP)])

        return carry

    lax.fori_loop(0, (NCHUNK_Q + NW - 1) // NW, body, 0)


def _p_gather(ha, hb, x4, row, col):
    k = functools.partial(
        pl.kernel,
        out_type=[
            jax.ShapeDtypeStruct((EQ, HALF), _f32),
            jax.ShapeDtypeStruct((EQ, HALF), _f32),
            jax.ShapeDtypeStruct((EQ * CDP,), _f32),
        ],
        mesh=_mesh(),
        scratch_types=[
            pltpu.VMEM((CH,), jnp.int32),
            pltpu.VMEM((CH,), jnp.int32),
            pltpu.VMEM((N * CD,), _f32),
            pltpu.VMEM((CH, HALF), _f32),
            pltpu.VMEM((CH, HALF), _f32),
            pltpu.VMEM((CH * CDP,), _f32),
            pltpu.SemaphoreType.DMA,
            pltpu.SemaphoreType.DMA,
        ],
        compiler_params=pltpu.CompilerParams(needs_layout_passes=False),
    )
    return k(_gather_body)(ha, hb, x4, row, col)


def _edge_body(ga, gb, cd8, ea, dmat, wrv, be1v, we2, be2v, wc1, bc1v,
               wc2t, msg2, cw):
    rad = cd8[:, CD:CD + 1]
    ua = lax.bitcast_convert_type(ga[...], jnp.int32)
    ub = lax.bitcast_convert_type(gb[...], jnp.int32)
    hi_mask = jnp.int32(-65536)
    s_e = (lax.bitcast_convert_type(jnp.left_shift(ua, 16), _f32)
           + lax.bitcast_convert_type(jnp.left_shift(ub, 16), _f32))
    s_o = (lax.bitcast_convert_type(ua & hi_mask, _f32)
           + lax.bitcast_convert_type(ub & hi_mask, _f32))
    pre = (jnp.concatenate([s_e, s_o], axis=1)
           + jnp.dot(ea[...], dmat[...], preferred_element_type=_f32)
           + rad * wrv[...] + be1v[...])
    m1 = _silu(pre).astype(jnp.bfloat16)
    msg = _silu(jnp.dot(m1, we2[...], preferred_element_type=_f32) + be2v[...])
    msg2[0, :, :] = msg[:, :HALF]
    msg2[1, :, :] = msg[:, HALF:]
    msgb = msg.astype(jnp.bfloat16)
    t = _silu(jnp.dot(msgb, wc1[...], preferred_element_type=_f32) + bc1v[...])
    cw[0] = lax.dot_general(wc2t[...], t, (((1,), (1,)), ((), ())),
                            preferred_element_type=_f32)


def _p_edge(ga, gb, cd8, ea, dmat, wrv, be1v, we2, be2v, wc1, bc1v, wc2t):
    full = lambda shape: pl.BlockSpec(shape, lambda i: tuple(0 for _ in shape))
    return pl.pallas_call(
        _edge_body,
        grid=(G_Q,),
        in_specs=[
            pl.BlockSpec((EB, HALF), lambda i: (i, 0)),
            pl.BlockSpec((EB, HALF), lambda i: (i, 0)),
            pl.BlockSpec((EB, CDP), lambda i: (i, 0)),
            pl.BlockSpec((EB, EA), lambda i: (i, 0)),
            full((EA, M)),
            full((1, M)),
            full((1, M)),
            full((M, M)),
            full((1, M)),
            full((M, M)),
            full((1, M)),
            full((1, M)),
        ],
        out_specs=[
            pl.BlockSpec((2, EB, HALF), lambda i: (0, i, 0)),
            pl.BlockSpec((1, 1, EB), lambda i: (i, 0, 0)),
        ],
        out_shape=[
            jax.ShapeDtypeStruct((2, EQ, HALF), _f32),
            jax.ShapeDtypeStruct((G_Q, 1, EB), _f32),
        ],
    )(ga, gb, cd8, ea, dmat, wrv, be1v, we2, be2v, wc1, bc1v, wc2t)


def _scatter_msg_body(msgf, row, prev, magg2, acc, idx_m, msg_buf):
    cid = lax.axis_index("c")
    sid = lax.axis_index("s")
    rpt = NP // NS

    pltpu.sync_copy(prev.at[pl.ds(cid * NP + sid * rpt, rpt)],
                    acc.at[pl.ds(sid * rpt, rpt)])

    plsc.subcore_barrier()

    def mbody(i, carry):
        c = sid + i * NS

        @pl.when(c < NCHUNK_Q)
        def _():
            e0 = c * CH
            pltpu.sync_copy(row.at[pl.ds(e0, CH)], idx_m)
            pltpu.sync_copy(msgf.at[pl.ds(cid * EQ + e0, CH)], msg_buf)
            pltpu.sync_copy(msg_buf, acc.at[idx_m], add=True)

        return carry

    lax.fori_loop(0, (NCHUNK_Q + NS - 1) // NS, mbody, 0)

    plsc.subcore_barrier()

    pltpu.sync_copy(acc.at[pl.ds(sid * rpt, rpt)],
                    magg2.at[pl.ds(cid * NP + sid * rpt, rpt)])


def _p_scatter_msg(msgf, row, prev):
    k = functools.partial(
        pl.kernel,
        out_type=jax.ShapeDtypeStruct((2 * NP, HALF), _f32),
        mesh=_mesh(),
        scratch_types=[
            pltpu.VMEM_SHARED((NP, HALF), _f32),
            pltpu.VMEM((CH,), jnp.int32),
            pltpu.VMEM((CH, HALF), _f32),
        ],
        compiler_params=pltpu.CompilerParams(needs_layout_passes=False),
    )
    return k(_scatter_msg_body)(msgf, row, prev)


def _scatter_x_body(cwf, cd8, row, zx, xp32, idx_w, cw_buf, cd_buf, accx):
    cid = lax.axis_index("c")
    sid = lax.axis_index("s")
    wid = sid * NC + cid

    pltpu.sync_copy(zx, accx)

    def wbody(i, carry):
        c = wid + i * NW

        @pl.when(c < NCHUNK)
        def _():
            e0 = c * CH
            pltpu.sync_copy(row.at[pl.ds(e0, CH)], idx_w)
            pltpu.sync_copy(cwf.at[pl.ds(e0, CH)], cw_buf)
            pltpu.sync_copy(cd8.at[pl.ds(e0 * CDP, CH * CDP)], cd_buf)

            def gbody(g, carry2):
                base = g * 16
                l8 = (lax.iota(jnp.int32, 16) + base) * CDP
                rv = idx_w[pl.ds(base, 16)] * CD
                cwv = cw_buf[pl.ds(base, 16)]
                for cc in range(CD):
                    dv = plsc.load_gather(cd_buf, [l8 + cc])
                    plsc.addupdate_scatter(accx, [rv + cc], dv * cwv)
                return carry2

            lax.fori_loop(0, CH // 16, gbody, 0)

        return carry

    lax.fori_loop(0, (NCHUNK + NW - 1) // NW, wbody, 0)

    for g in range(XACC // XB):
        pltpu.sync_copy(accx.at[pl.ds(g * XB, XB)],
                        xp32.at[pl.ds((g * NW + wid) * XB, XB)])


def _p_scatter_x(cwf, cd8, row, zx):
    k = functools.partial(
        pl.kernel,
        out_type=jax.ShapeDtypeStruct((NW * XACC,), _f32),
        mesh=_mesh(),
        scratch_types=[
            pltpu.VMEM((CH,), jnp.int32),
            pltpu.VMEM((CH,), _f32),
            pltpu.VMEM((CH * CDP,), _f32),
            pltpu.VMEM((XACC,), _f32),
        ],
        compiler_params=pltpu.CompilerParams(needs_layout_passes=False),
    )
    return k(_scatter_x_body)(cwf, cd8, row, zx)


def _final_h_body(h_ref, ml, mr, wh, wml, wmr, bf1v, wf2, bf2v, out):
    hf = (jnp.dot(h_ref[...], wh[...], preferred_element_type=_f32)
          + jnp.dot(ml[...], wml[...], preferred_element_type=_f32)
          + jnp.dot(mr[...], wmr[...], preferred_element_type=_f32)
          + bf1v[...])
    hf = _silu(hf)
    out[...] = jnp.dot(hf, wf2[...], preferred_element_type=_f32) + bf2v[...]


def _p_final_h(h, ml, mr, wh, wml, wmr, bf1v, wf2, bf2v):
    full = lambda shape: pl.BlockSpec(shape, lambda i: tuple(0 for _ in shape))
    return pl.pallas_call(
        _final_h_body,
        grid=(GN,),
        in_specs=[
            pl.BlockSpec((NB, F), lambda i: (i, 0)),
            pl.BlockSpec((NB, HALF), lambda i: (i, 0)),
            pl.BlockSpec((NB, HALF), lambda i: (i, 0)),
            full((F, M)),
            full((HALF, M)),
            full((HALF, M)),
            full((1, M)),
            full((M, F)),
            full((1, F)),
        ],
        out_specs=pl.BlockSpec((NB, F), lambda i: (i, 0)),
        out_shape=jax.ShapeDtypeStruct((N, F), _f32),
    )(h, ml, mr, wh, wml, wmr, bf1v, wf2, bf2v)


def _final_x_body(xv, xps, xout):
    xout[0, 0, :] = xv[0, 0, :] + jnp.sum(xps[0], axis=0)


def _p_final_x(xflat, xp32):
    return pl.pallas_call(
        _final_x_body,
        grid=(GN,),
        in_specs=[
            pl.BlockSpec((1, 1, XB), lambda i: (i, 0, 0)),
            pl.BlockSpec((1, NW, XB), lambda i: (i, 0, 0)),
        ],
        out_specs=pl.BlockSpec((1, 1, XB), lambda i: (i, 0, 0)),
        out_shape=jax.ShapeDtypeStruct((GN, 1, XB), _f32),
    )(xflat, xp32)


def kernel(h, x, edge_index, edge_attribute, We1, be1, We2, be2,
           Wf1, bf1, Wf2, bf2, Wc1, bc1, Wc2):
    row = edge_index[0].astype(jnp.int32)
    col = edge_index[1].astype(jnp.int32)

    ab = jnp.concatenate([We1[:F], We1[F:2 * F]], axis=1)
    perm = jnp.concatenate([jnp.arange(0, M, 2), jnp.arange(1, M, 2)])
    wrv = We1[2 * F][perm].reshape(1, M)
    dmat = We1[2 * F + 1:][:, perm]

    ha, hb = _p_pre(h, ab)
    pack = lambda t: lax.bitcast_convert_type(
        t.astype(jnp.bfloat16).reshape(N, HALF, 2), _f32)
    hap, hbp = pack(ha), pack(hb)
    xflat4 = x.reshape(N * CD)

    we2b = We2[perm].astype(jnp.bfloat16)
    wc1b = Wc1.astype(jnp.bfloat16)
    be1v = be1[perm].reshape(1, M)

    magg2 = jnp.zeros((2 * NP, HALF), _f32)
    cw_parts, cd_parts = [], []
    for q in range(Q):
        row_q = lax.slice(row, (q * EQ,), ((q + 1) * EQ,))
        col_q = lax.slice(col, (q * EQ,), ((q + 1) * EQ,))
        ea_q = lax.slice(edge_attribute, (q * EQ, 0), ((q + 1) * EQ, EA))
        ga, gb, cd8 = _p_gather(hap, hbp, xflat4, row_q, col_q)
        msg2, cwg = _p_edge(
            ga, gb, cd8.reshape(EQ, CDP), ea_q, dmat, wrv,
            be1v, we2b, be2.reshape(1, M), wc1b,
            bc1.reshape(1, M), Wc2.reshape(1, M))
        magg2 = _p_scatter_msg(msg2.reshape(2 * EQ, HALF), row_q, magg2)
        cw_parts.append(cwg.reshape(EQ))
        cd_parts.append(cd8)
    xp32 = _p_scatter_x(jnp.concatenate(cw_parts), jnp.concatenate(cd_parts),
                        row, jnp.zeros((XACC,), _f32))

    h_updated = _p_final_h(
        h, magg2[:N], magg2[NP:NP + N],
        Wf1[:F], Wf1[F:F + HALF], Wf1[F + HALF:], bf1.reshape(1, M), Wf2,
        bf2.reshape(1, F))
    xflat = jnp.pad(x.reshape(N * CD), (0, XACC - N * CD)).reshape(GN, 1, XB)
    xf = _p_final_x(xflat, xp32.reshape(GN, NW, XB))
    x_updated = xf.reshape(XACC)[:N * CD].reshape(N, CD)
    return (h_updated, x_updated)
